# Initial kernel scaffold; baseline (speedup 1.0000x reference)
import functools

import jax
import jax.numpy as jnp
from jax.experimental import pallas as pl
from jax.experimental.pallas import tpu as pltpu


LAYERS = 2


def _msg_kernel(h_ref, wm_ref, bm_ref, msg_ref):
    B = h_ref.shape[0]
    M = wm_ref.shape[1]
    for b in range(B):
        mb = jnp.dot(h_ref[b], wm_ref[...], preferred_element_type=jnp.float32)
        msg_ref[:, b * M:(b + 1) * M] = jnp.maximum(mb + bm_ref[...], 0.0)


def _layer_kernel(a_ref, msg_ref, h_ref, wuh_ref, wua_ref, bu_ref,
                  hout_ref, acc_ref, deg_ref, *, bj, msg_dim):
    j = pl.program_id(1)
    nj = pl.num_programs(1)
    a = a_ref[...]
    rs = jnp.sum(a, axis=1, keepdims=True)
    mblk = msg_ref[pl.ds(j * bj, bj), :]
    part = jax.lax.dot_general(
        a, mblk, dimension_numbers=(((1,), (0,)), ((), ())),
        preferred_element_type=jnp.float32)

    @pl.when(j == 0)
    def _init():
        acc_ref[...] = part
        deg_ref[...] = rs

    @pl.when(j > 0)
    def _accum():
        acc_ref[...] += part
        deg_ref[...] += rs

    @pl.when(j == nj - 1)
    def _finish():
        inv = 1.0 / jnp.clip(deg_ref[...], 1.0, None)
        agg = acc_ref[...] * inv
        B = h_ref.shape[0]
        for b in range(B):
            hb = h_ref[b]
            ab = agg[:, b * msg_dim:(b + 1) * msg_dim]
            u = (jnp.dot(hb, wuh_ref[...], preferred_element_type=jnp.float32)
                 + jnp.dot(ab, wua_ref[...], preferred_element_type=jnp.float32)
                 + bu_ref[...])
            hout_ref[b] = jnp.maximum(u, 0.0)


def _readout_kernel(h_ref, wo_ref, bo_ref, out_ref, acc_ref, *, n_nodes):
    i = pl.program_id(0)
    ni = pl.num_programs(0)
    ps = jnp.sum(h_ref[...], axis=1)

    @pl.when(i == 0)
    def _init():
        acc_ref[...] = ps

    @pl.when(i > 0)
    def _accum():
        acc_ref[...] += ps

    @pl.when(i == ni - 1)
    def _finish():
        pooled = acc_ref[...] * (1.0 / n_nodes)
        out_ref[...] = (jnp.dot(pooled, wo_ref[...],
                                preferred_element_type=jnp.float32)
                        + bo_ref[...])


def _pick_block(n, pref):
    for p in pref:
        if n % p == 0:
            return p
    return n


@jax.jit
def kernel(h_init, A, W_msg, b_msg, W_upd, b_upd, W_out, b_out):
    B, N, H = h_init.shape
    M = W_msg.shape[1]
    OUT = W_out.shape[1]
    BM = B * M

    bm2 = b_msg.reshape(1, M)
    bu2 = b_upd.reshape(1, H)
    bo2 = b_out.reshape(1, OUT)
    wuh = W_upd[:H, :]
    wua = W_upd[H:, :]

    bi = _pick_block(N, (1000, 500, 250))
    bj = _pick_block(N, (2000, 1000, 500))
    bjm = _pick_block(N, (2000, 1000, 500))

    msg_call = pl.pallas_call(
        _msg_kernel,
        grid=(N // bjm,),
        in_specs=[
            pl.BlockSpec((B, bjm, H), lambda j: (0, j, 0)),
            pl.BlockSpec((H, M), lambda j: (0, 0)),
            pl.BlockSpec((1, M), lambda j: (0, 0)),
        ],
        out_specs=pl.BlockSpec((bjm, BM), lambda j: (j, 0)),
        out_shape=jax.ShapeDtypeStruct((N, BM), jnp.float32),
    )

    layer_call = pl.pallas_call(
        functools.partial(_layer_kernel, bj=bj, msg_dim=M),
        grid=(N // bi, N // bj),
        in_specs=[
            pl.BlockSpec((bi, bj), lambda i, j: (i, j)),
            pl.BlockSpec((N, BM), lambda i, j: (0, 0)),
            pl.BlockSpec((B, bi, H), lambda i, j: (0, i, 0)),
            pl.BlockSpec((H, H), lambda i, j: (0, 0)),
            pl.BlockSpec((M, H), lambda i, j: (0, 0)),
            pl.BlockSpec((1, H), lambda i, j: (0, 0)),
        ],
        out_specs=pl.BlockSpec((B, bi, H), lambda i, j: (0, i, 0)),
        out_shape=jax.ShapeDtypeStruct((B, N, H), jnp.float32),
        scratch_shapes=[
            pltpu.VMEM((bi, BM), jnp.float32),
            pltpu.VMEM((bi, 1), jnp.float32),
        ],
    )

    readout_call = pl.pallas_call(
        functools.partial(_readout_kernel, n_nodes=float(N)),
        grid=(N // bi,),
        in_specs=[
            pl.BlockSpec((B, bi, H), lambda i: (0, i, 0)),
            pl.BlockSpec((H, OUT), lambda i: (0, 0)),
            pl.BlockSpec((1, OUT), lambda i: (0, 0)),
        ],
        out_specs=pl.BlockSpec((B, OUT), lambda i: (0, 0)),
        out_shape=jax.ShapeDtypeStruct((B, OUT), jnp.float32),
        scratch_shapes=[
            pltpu.VMEM((B, OUT), jnp.float32),
        ],
    )

    h = h_init
    for _ in range(LAYERS):
        msg = msg_call(h, W_msg, bm2)
        h = layer_call(A, msg, h, wuh, wua, bu2)
    return readout_call(h, W_out, bo2)


# speedup vs baseline: 1.6564x; 1.6564x over previous
import functools

import jax
import jax.numpy as jnp
from jax.experimental import pallas as pl
from jax.experimental.pallas import tpu as pltpu


LAYERS = 2


def _msg_kernel(h_ref, wm_ref, bm_ref, msg_ref):
    B = h_ref.shape[0]
    M = wm_ref.shape[1]
    for b in range(B):
        mb = jnp.dot(h_ref[b], wm_ref[...], preferred_element_type=jnp.float32)
        msg_ref[:, b * M:(b + 1) * M] = jnp.maximum(mb + bm_ref[...], 0.0)


def _layer_kernel(a_ref, msg_ref, h_ref, wuh_ref, wua_ref, bu_ref,
                  hout_ref, *, msg_dim):
    a = a_ref[...]
    deg = jnp.sum(a, axis=1, keepdims=True)
    acc = jax.lax.dot_general(
        a, msg_ref[...], dimension_numbers=(((1,), (0,)), ((), ())),
        preferred_element_type=jnp.float32)
    inv = 1.0 / jnp.clip(deg, 1.0, None)
    agg = acc * inv
    B = h_ref.shape[0]
    for b in range(B):
        hb = h_ref[b]
        ab = agg[:, b * msg_dim:(b + 1) * msg_dim]
        u = (jnp.dot(hb, wuh_ref[...], preferred_element_type=jnp.float32)
             + jnp.dot(ab, wua_ref[...], preferred_element_type=jnp.float32)
             + bu_ref[...])
        hout_ref[b] = jnp.maximum(u, 0.0)


def _readout_kernel(h_ref, wo_ref, bo_ref, out_ref, acc_ref, *, n_nodes):
    i = pl.program_id(0)
    ni = pl.num_programs(0)
    ps = jnp.sum(h_ref[...], axis=1)

    @pl.when(i == 0)
    def _init():
        acc_ref[...] = ps

    @pl.when(i > 0)
    def _accum():
        acc_ref[...] += ps

    @pl.when(i == ni - 1)
    def _finish():
        pooled = acc_ref[...] * (1.0 / n_nodes)
        out_ref[...] = (jnp.dot(pooled, wo_ref[...],
                                preferred_element_type=jnp.float32)
                        + bo_ref[...])


def _pick_block(n, pref):
    for p in pref:
        if n % p == 0:
            return p
    return n


@jax.jit
def kernel(h_init, A, W_msg, b_msg, W_upd, b_upd, W_out, b_out):
    B, N, H = h_init.shape
    M = W_msg.shape[1]
    OUT = W_out.shape[1]
    BM = B * M

    bm2 = b_msg.reshape(1, M)
    bu2 = b_upd.reshape(1, H)
    bo2 = b_out.reshape(1, OUT)
    wuh = W_upd[:H, :]
    wua = W_upd[H:, :]

    bi = _pick_block(N, (400, 200, 80, 8))
    bjm = _pick_block(N, (2000, 1000, 500))

    msg_call = pl.pallas_call(
        _msg_kernel,
        grid=(N // bjm,),
        in_specs=[
            pl.BlockSpec((B, bjm, H), lambda j: (0, j, 0)),
            pl.BlockSpec((H, M), lambda j: (0, 0)),
            pl.BlockSpec((1, M), lambda j: (0, 0)),
        ],
        out_specs=pl.BlockSpec((bjm, BM), lambda j: (j, 0)),
        out_shape=jax.ShapeDtypeStruct((N, BM), jnp.float32),
    )

    layer_call = pl.pallas_call(
        functools.partial(_layer_kernel, msg_dim=M),
        grid=(N // bi,),
        in_specs=[
            pl.BlockSpec((bi, N), lambda i: (i, 0)),
            pl.BlockSpec((N, BM), lambda i: (0, 0)),
            pl.BlockSpec((B, bi, H), lambda i: (0, i, 0)),
            pl.BlockSpec((H, H), lambda i: (0, 0)),
            pl.BlockSpec((M, H), lambda i: (0, 0)),
            pl.BlockSpec((1, H), lambda i: (0, 0)),
        ],
        out_specs=pl.BlockSpec((B, bi, H), lambda i: (0, i, 0)),
        out_shape=jax.ShapeDtypeStruct((B, N, H), jnp.float32),
    )

    readout_call = pl.pallas_call(
        functools.partial(_readout_kernel, n_nodes=float(N)),
        grid=(N // bi,),
        in_specs=[
            pl.BlockSpec((B, bi, H), lambda i: (0, i, 0)),
            pl.BlockSpec((H, OUT), lambda i: (0, 0)),
            pl.BlockSpec((1, OUT), lambda i: (0, 0)),
        ],
        out_specs=pl.BlockSpec((B, OUT), lambda i: (0, 0)),
        out_shape=jax.ShapeDtypeStruct((B, OUT), jnp.float32),
        scratch_shapes=[
            pltpu.VMEM((B, OUT), jnp.float32),
        ],
    )

    h = h_init
    for _ in range(LAYERS):
        msg = msg_call(h, W_msg, bm2)
        h = layer_call(A, msg, h, wuh, wua, bu2)
    return readout_call(h, W_out, bo2)
